# baseline (speedup 1.0000x reference)
import functools

import jax
import jax.numpy as jnp
from jax import lax
from jax.experimental import pallas as pl
from jax.experimental.pallas import tpu as pltpu
from jax.experimental.pallas import tpu_sc as plsc

NB = 4
BB = 512


def _transpose_kernel(x_ref, o_ref):
    o_ref[...] = jnp.transpose(x_ref[...], (0, 2, 1))


def kernel(x, w):
    b, s = x.shape
    v, d = w.shape
    h = s // 2
    NW = 32
    bpw = b // NW
    n_chunks = bpw // NB
    assert bpw * NW == b and n_chunks * NB == bpw and n_chunks % 2 == 0
    xe = x[:, 0::2]
    xo = x[:, 1::2]
    mesh = plsc.VectorSubcoreMesh(core_axis_name="c", subcore_axis_name="s")
    NC = mesh.num_cores

    @functools.partial(
        pl.kernel,
        out_type=jax.ShapeDtypeStruct((h, b, 2 * d), jnp.float32),
        mesh=mesh,
        scratch_types=[
            pltpu.VMEM((2, NB, h), jnp.int32),
            pltpu.VMEM((2, NB, h), jnp.int32),
            pltpu.VMEM((NB * s, d), jnp.float32),
            pltpu.VMEM((NB * s, d), jnp.float32),
            pltpu.SemaphoreType.DMA,
            pltpu.SemaphoreType.DMA,
        ],
        compiler_params=pltpu.CompilerParams(use_tc_tiling_on_sc=False),
    )
    def gather_stage(table_hbm, xe_hbm, xo_hbm, out_hbm, idx0, idx1,
                     rows0, rows1, gsem0, gsem1):
        wid = lax.axis_index("s") * NC + lax.axis_index("c")
        base = wid * bpw

        def fire(ci, idx_v, rows_v, gsem):
            b0 = pl.multiple_of(base + ci * NB, NB)
            pltpu.sync_copy(xe_hbm.at[pl.ds(b0, NB)], idx_v.at[0])
            pltpu.sync_copy(xo_hbm.at[pl.ds(b0, NB)], idx_v.at[1])
            for j in range(NB):
                pltpu.async_copy(
                    table_hbm.at[idx_v.at[0, j]],
                    rows_v.at[pl.ds(j * s, h)],
                    gsem,
                )
                pltpu.async_copy(
                    table_hbm.at[idx_v.at[1, j]],
                    rows_v.at[pl.ds(j * s + h, h)],
                    gsem,
                )

        def drain_and_write(ci, rows_v, gsem):
            pltpu.make_async_copy(
                table_hbm.at[pl.ds(0, NB * s)], rows_v, gsem
            ).wait()
            b0 = pl.multiple_of(base + ci * NB, NB)
            for j in range(NB):
                pltpu.sync_copy(
                    rows_v.at[pl.ds(j * s, h)],
                    out_hbm.at[pl.ds(0, h), b0 + j, pl.ds(0, d)],
                )
                pltpu.sync_copy(
                    rows_v.at[pl.ds(j * s + h, h)],
                    out_hbm.at[pl.ds(0, h), b0 + j, pl.ds(d, d)],
                )

        fire(0, idx0, rows0, gsem0)

        def pair(j, carry):
            ca = 2 * j
            fire(ca + 1, idx1, rows1, gsem1)
            drain_and_write(ca, rows0, gsem0)

            @pl.when(j < n_chunks // 2 - 1)
            def _():
                fire(ca + 2, idx0, rows0, gsem0)

            drain_and_write(ca + 1, rows1, gsem1)
            return carry

        lax.fori_loop(0, n_chunks // 2, pair, 0)

    tbm = gather_stage(w, xe, xo)

    o2 = pl.pallas_call(
        _transpose_kernel,
        grid=(h,),
        in_specs=[
            pl.BlockSpec((1, b, 2 * d), lambda t: (t, 0, 0)),
        ],
        out_specs=pl.BlockSpec((1, 2 * d, b), lambda t: (t, 0, 0)),
        out_shape=jax.ShapeDtypeStruct((h, 2 * d, b), jnp.float32),
    )(tbm)

    return jnp.transpose(o2, (2, 0, 1)).reshape(b, s, d)


# speedup vs baseline: 1.0289x; 1.0289x over previous
import functools

import jax
import jax.numpy as jnp
from jax import lax
from jax.experimental import pallas as pl
from jax.experimental.pallas import tpu as pltpu
from jax.experimental.pallas import tpu_sc as plsc

NB = 4
BB = 512


def _transpose_kernel(x_ref, o_ref):
    o_ref[...] = jnp.transpose(x_ref[...], (0, 2, 1))


def kernel(x, w):
    b, s = x.shape
    v, d = w.shape
    h = s // 2
    NW = 32
    bpw = b // NW
    n_chunks = bpw // NB
    assert bpw * NW == b and n_chunks * NB == bpw and n_chunks % 2 == 0
    xeo = x.reshape(b, h, 2).transpose(0, 2, 1).reshape(b, s)
    mesh = plsc.VectorSubcoreMesh(core_axis_name="c", subcore_axis_name="s")
    NC = mesh.num_cores

    @functools.partial(
        pl.kernel,
        out_type=jax.ShapeDtypeStruct((h, b, 2 * d), jnp.float32),
        mesh=mesh,
        scratch_types=[
            pltpu.VMEM((NB, s), jnp.int32),
            pltpu.VMEM((NB, s), jnp.int32),
            pltpu.VMEM((NB * s, d), jnp.float32),
            pltpu.VMEM((NB * s, d), jnp.float32),
            pltpu.SemaphoreType.DMA,
            pltpu.SemaphoreType.DMA,
        ],
        compiler_params=pltpu.CompilerParams(use_tc_tiling_on_sc=False),
    )
    def gather_stage(table_hbm, xeo_hbm, out_hbm, idx0, idx1, rows0, rows1,
                     gsem0, gsem1):
        wid = lax.axis_index("s") * NC + lax.axis_index("c")
        base = wid * bpw

        def fire(ci, idx_v, rows_v, gsem):
            b0 = pl.multiple_of(base + ci * NB, NB)
            pltpu.sync_copy(xeo_hbm.at[pl.ds(b0, NB)], idx_v)
            for j in range(NB):
                pltpu.async_copy(
                    table_hbm.at[idx_v.at[j]],
                    rows_v.at[pl.ds(j * s, s)],
                    gsem,
                )

        def drain_and_write(ci, rows_v, gsem):
            pltpu.make_async_copy(
                table_hbm.at[pl.ds(0, NB * s)], rows_v, gsem
            ).wait()
            b0 = pl.multiple_of(base + ci * NB, NB)
            for j in range(NB):
                pltpu.sync_copy(
                    rows_v.at[pl.ds(j * s, h)],
                    out_hbm.at[pl.ds(0, h), b0 + j, pl.ds(0, d)],
                )
                pltpu.sync_copy(
                    rows_v.at[pl.ds(j * s + h, h)],
                    out_hbm.at[pl.ds(0, h), b0 + j, pl.ds(d, d)],
                )

        fire(0, idx0, rows0, gsem0)

        def pair(j, carry):
            ca = 2 * j
            fire(ca + 1, idx1, rows1, gsem1)
            drain_and_write(ca, rows0, gsem0)

            @pl.when(j < n_chunks // 2 - 1)
            def _():
                fire(ca + 2, idx0, rows0, gsem0)

            drain_and_write(ca + 1, rows1, gsem1)
            return carry

        lax.fori_loop(0, n_chunks // 2, pair, 0)

    tbm = gather_stage(w, xeo)

    o2 = pl.pallas_call(
        _transpose_kernel,
        grid=(h,),
        in_specs=[
            pl.BlockSpec((1, b, 2 * d), lambda t: (t, 0, 0)),
        ],
        out_specs=pl.BlockSpec((1, 2 * d, b), lambda t: (t, 0, 0)),
        out_shape=jax.ShapeDtypeStruct((h, 2 * d, b), jnp.float32),
    )(tbm)

    return jnp.transpose(o2, (2, 0, 1)).reshape(b, s, d)
